# baseline (speedup 1.0000x reference)
import functools

import jax
import jax.numpy as jnp
from jax import lax
from jax.experimental import pallas as pl
from jax.experimental.pallas import tpu as pltpu
from jax.experimental.pallas import tpu_sc as plsc

N_DAYS = 16384
N_TIMES = 390
F = 8
WINDOW = 60
L = 16
NC, NS = 2, 16
NW = NC * NS
N_ROWS = WINDOW * F
ROWS_PER_W = N_ROWS // NW


def _make_gather(batch):
    n_vec = batch // L
    mesh = plsc.VectorSubcoreMesh(
        core_axis_name="c", subcore_axis_name="s",
        num_cores=NC, num_subcores=NS)

    @functools.partial(
        pl.kernel,
        mesh=mesh,
        out_type=jax.ShapeDtypeStruct((N_ROWS, batch), jnp.float32),
        scratch_types=[
            pltpu.VMEM((batch,), jnp.int32),
            pltpu.VMEM((batch,), jnp.float32),
            pltpu.VMEM((batch,), jnp.float32),
            pltpu.VMEM((batch,), jnp.float32),
            pltpu.VMEM((batch,), jnp.float32),
            pltpu.VMEM((batch,), jnp.float32),
            pltpu.VMEM((batch,), jnp.float32),
            pltpu.VMEM((L,), jnp.int32),
            pltpu.SemaphoreType.DMA,
            pltpu.SemaphoreType.DMA,
        ],
        compiler_params=pltpu.CompilerParams(needs_layout_passes=False),
    )
    def k(table, didx, row0, out, didx_v, slab_a, slab_b, slab_c, slab_d,
          orow_a, orow_b, r0_v, ld_sem, st_sem):
        wid = lax.axis_index("s") * NC + lax.axis_index("c")
        s0 = wid * ROWS_PER_W
        pltpu.sync_copy(didx.at[pl.ds(0, batch)], didx_v)
        pltpu.sync_copy(row0, r0_v)
        r0s = jnp.min(r0_v[...])
        DEPTH = 4
        half = batch // 2
        slabs = [slab_a, slab_b, slab_c, slab_d]
        orows = [orow_a, orow_b]
        copies = [None] * DEPTH
        stores = [None, None]

        quart = batch // 4

        def load_row(row, buf, j):
            return [
                pltpu.async_copy(
                    table.at[row, pl.ds(q * quart, quart)],
                    buf.at[pl.ds(q * quart, quart)], ld_sem)
                for q in range(4)
            ]

        for j in range(DEPTH - 1):
            copies[j] = load_row(r0s + s0 + j, slabs[j], j)
        for i in range(ROWS_PER_W):
            cur = i % DEPTH
            for cp in copies[cur]:
                cp.wait()
            if i + DEPTH - 1 < ROWS_PER_W:
                nxt = (i + DEPTH - 1) % DEPTH
                copies[nxt] = load_row(
                    r0s + s0 + i + DEPTH - 1, slabs[nxt], nxt)
            slab = slabs[cur]
            orow_v = orows[i % 2]
            if stores[i % 2] is not None:
                stores[i % 2].wait()

            @plsc.parallel_loop(0, n_vec, 1, unroll=8)
            def body(v):
                idx = didx_v[pl.ds(v * L, L)]
                orow_v[pl.ds(v * L, L)] = plsc.load_gather(slab, [idx])

            stores[i % 2] = pltpu.async_copy(orow_v, out.at[s0 + i], st_sem)
        for s in stores:
            if s is not None:
                s.wait()

    return k


def kernel(price_data, date_idx, time_idx):
    batch = date_idx.shape[0]
    table = jnp.transpose(price_data, (1, 2, 0)).reshape(N_TIMES * F, N_DAYS)
    row0 = (time_idx - WINDOW) * F
    row0_arr = jnp.full((L,), row0, dtype=jnp.int32)
    didx = date_idx.astype(jnp.int32)
    out = _make_gather(batch)(table, didx, row0_arr)
    return jnp.transpose(out.reshape(WINDOW, F, batch), (2, 0, 1))


# speedup vs baseline: 1.0080x; 1.0080x over previous
import functools

import jax
import jax.numpy as jnp
from jax import lax
from jax.experimental import pallas as pl
from jax.experimental.pallas import tpu as pltpu
from jax.experimental.pallas import tpu_sc as plsc

N_DAYS = 16384
N_TIMES = 390
F = 8
WINDOW = 60
L = 16
NC, NS = 2, 16
NW = NC * NS
N_ROWS = WINDOW * F
ROWS_PER_W = N_ROWS // NW


def _make_gather(batch):
    n_vec = batch // L
    mesh = plsc.VectorSubcoreMesh(
        core_axis_name="c", subcore_axis_name="s",
        num_cores=NC, num_subcores=NS)

    @functools.partial(
        pl.kernel,
        mesh=mesh,
        out_type=jax.ShapeDtypeStruct((N_ROWS, batch), jnp.float32),
        scratch_types=[
            pltpu.VMEM((batch,), jnp.int32),
            pltpu.VMEM((batch,), jnp.float32),
            pltpu.VMEM((batch,), jnp.float32),
            pltpu.VMEM((batch,), jnp.float32),
            pltpu.VMEM((batch,), jnp.float32),
            pltpu.VMEM((batch,), jnp.float32),
            pltpu.VMEM((batch,), jnp.float32),
            pltpu.VMEM((L,), jnp.int32),
            pltpu.SemaphoreType.DMA,
            pltpu.SemaphoreType.DMA,
        ],
        compiler_params=pltpu.CompilerParams(needs_layout_passes=False),
    )
    def k(table, didx, row0, out, didx_v, slab_a, slab_b, slab_c, slab_d,
          orow_a, orow_b, r0_v, ld_sem, st_sem):
        wid = lax.axis_index("s") * NC + lax.axis_index("c")
        s0 = wid * ROWS_PER_W
        pltpu.sync_copy(didx.at[pl.ds(0, batch)], didx_v)
        pltpu.sync_copy(row0, r0_v)
        r0s = jnp.min(r0_v[...])
        DEPTH = 4
        half = batch // 2
        slabs = [slab_a, slab_b, slab_c, slab_d]
        orows = [orow_a, orow_b]
        copies = [None] * DEPTH
        stores = [None, None]

        def load_row(row, buf, j):
            return [
                pltpu.async_copy(
                    table.at[row, pl.ds(0, half)],
                    buf.at[pl.ds(0, half)], ld_sem),
                pltpu.async_copy(
                    table.at[row, pl.ds(half, half)],
                    buf.at[pl.ds(half, half)], ld_sem),
            ]

        for j in range(DEPTH - 1):
            copies[j] = load_row(r0s + s0 + j, slabs[j], j)
        for i in range(ROWS_PER_W):
            cur = i % DEPTH
            for cp in copies[cur]:
                cp.wait()
            if i + DEPTH - 1 < ROWS_PER_W:
                nxt = (i + DEPTH - 1) % DEPTH
                copies[nxt] = load_row(
                    r0s + s0 + i + DEPTH - 1, slabs[nxt], nxt)
            slab = slabs[cur]
            orow_v = orows[i % 2]
            if stores[i % 2] is not None:
                stores[i % 2].wait()

            @plsc.parallel_loop(0, n_vec, 1, unroll=8)
            def body(v):
                idx = didx_v[pl.ds(v * L, L)]
                orow_v[pl.ds(v * L, L)] = plsc.load_gather(slab, [idx])

            stores[i % 2] = pltpu.async_copy(orow_v, out.at[s0 + i], st_sem)
        for s in stores:
            if s is not None:
                s.wait()

    return k


def kernel(price_data, date_idx, time_idx):
    batch = date_idx.shape[0]
    table = jnp.transpose(price_data, (1, 2, 0)).reshape(N_TIMES * F, N_DAYS)
    row0 = (time_idx - WINDOW) * F
    row0_arr = jnp.full((L,), row0, dtype=jnp.int32)
    didx = date_idx.astype(jnp.int32)
    out = _make_gather(batch)(table, didx, row0_arr)
    return jnp.transpose(out.reshape(WINDOW, F, batch), (2, 0, 1))


# speedup vs baseline: 1.0196x; 1.0115x over previous
import functools

import jax
import jax.numpy as jnp
from jax import lax
from jax.experimental import pallas as pl
from jax.experimental.pallas import tpu as pltpu
from jax.experimental.pallas import tpu_sc as plsc

N_DAYS = 16384
N_TIMES = 390
F = 8
WINDOW = 60
L = 16
NC, NS = 2, 16
NW = NC * NS
N_ROWS = WINDOW * F
ROWS_PER_W = N_ROWS // NW


def _make_gather(batch):
    n_vec = batch // L
    mesh = plsc.VectorSubcoreMesh(
        core_axis_name="c", subcore_axis_name="s",
        num_cores=NC, num_subcores=NS)

    @functools.partial(
        pl.kernel,
        mesh=mesh,
        out_type=jax.ShapeDtypeStruct((N_ROWS, batch), jnp.float32),
        scratch_types=[
            pltpu.VMEM((batch,), jnp.int32),
            pltpu.VMEM((batch,), jnp.float32),
            pltpu.VMEM((batch,), jnp.float32),
            pltpu.VMEM((batch,), jnp.float32),
            pltpu.VMEM((batch,), jnp.float32),
            pltpu.VMEM((batch,), jnp.float32),
            pltpu.VMEM((batch,), jnp.float32),
            pltpu.VMEM((L,), jnp.int32),
            pltpu.SemaphoreType.DMA,
            pltpu.SemaphoreType.DMA,
        ],
        compiler_params=pltpu.CompilerParams(needs_layout_passes=False),
    )
    def k(table, didx, row0, out, didx_v, slab_a, slab_b, slab_c,
          orow_a, orow_b, orow_c, r0_v, ld_sem, st_sem):
        wid = lax.axis_index("s") * NC + lax.axis_index("c")
        s0 = wid * ROWS_PER_W
        pltpu.sync_copy(didx.at[pl.ds(0, batch)], didx_v)
        pltpu.sync_copy(row0, r0_v)
        r0s = jnp.min(r0_v[...])
        DEPTH = 3
        half = batch // 2
        slabs = [slab_a, slab_b, slab_c]
        orows = [orow_a, orow_b, orow_c]
        copies = [None] * DEPTH
        stores = [None, None, None]

        def load_row(row, buf, j):
            return [
                pltpu.async_copy(
                    table.at[row, pl.ds(0, half)],
                    buf.at[pl.ds(0, half)], ld_sem),
                pltpu.async_copy(
                    table.at[row, pl.ds(half, half)],
                    buf.at[pl.ds(half, half)], ld_sem),
            ]

        for j in range(DEPTH - 1):
            copies[j] = load_row(r0s + s0 + j, slabs[j], j)
        for i in range(ROWS_PER_W):
            cur = i % DEPTH
            for cp in copies[cur]:
                cp.wait()
            if i + DEPTH - 1 < ROWS_PER_W:
                nxt = (i + DEPTH - 1) % DEPTH
                copies[nxt] = load_row(
                    r0s + s0 + i + DEPTH - 1, slabs[nxt], nxt)
            slab = slabs[cur]
            orow_v = orows[i % 3]
            if stores[i % 3] is not None:
                stores[i % 3].wait()

            @plsc.parallel_loop(0, n_vec, 1, unroll=8)
            def body(v):
                idx = didx_v[pl.ds(v * L, L)]
                orow_v[pl.ds(v * L, L)] = plsc.load_gather(slab, [idx])

            stores[i % 3] = pltpu.async_copy(orow_v, out.at[s0 + i], st_sem)
        for s in stores:
            if s is not None:
                s.wait()

    return k


def kernel(price_data, date_idx, time_idx):
    batch = date_idx.shape[0]
    table = jnp.transpose(price_data, (1, 2, 0)).reshape(N_TIMES * F, N_DAYS)
    row0 = (time_idx - WINDOW) * F
    row0_arr = jnp.full((L,), row0, dtype=jnp.int32)
    didx = date_idx.astype(jnp.int32)
    out = _make_gather(batch)(table, didx, row0_arr)
    return jnp.transpose(out.reshape(WINDOW, F, batch), (2, 0, 1))
